# baseline (speedup 1.0000x reference)
import functools

import jax
import jax.numpy as jnp
from jax import lax
from jax.experimental import pallas as pl
from jax.experimental.pallas import tpu as pltpu
from jax.experimental.pallas import tpu_sc as plsc

N = 10000
E = 320000
D_IN = 128
D_H1 = 512
D_H2 = 256

NC = 2
NS = 16
K = 80
NROWB = N // K

_MESH = plsc.VectorSubcoreMesh(core_axis_name="c", subcore_axis_name="s")


def _scale_rows(rows_v, w_v, n_feat16):
    dnums = lax.GatherDimensionNumbers(
        offset_dims=(), collapsed_slice_dims=(0,), start_index_map=(0,))

    @plsc.parallel_loop(0, K // 16)
    def group(j):
        w16 = w_v[pl.ds(j * 16, 16)]
        for l in range(16):
            wk = lax.gather(w16, jnp.full((16, 1), l, jnp.int32), dnums, (1,),
                            mode=lax.GatherScatterMode.PROMISE_IN_BOUNDS)
            k = j * 16 + l
            for d in range(n_feat16):
                sl = pl.ds(d * 16, 16)
                rows_v[k, sl] = rows_v[k, sl] * wk


def _zero_fill(buf, nrows, width16):
    def row(i, carry):
        for d in range(width16):
            buf[i, pl.ds(d * 16, 16)] = jnp.zeros((16,), jnp.float32)
        return carry
    lax.fori_loop(0, nrows, row, 0, unroll=False)


def _fill_iota(idx_v, base):
    for j in range(K // 16):
        idx_v[pl.ds(j * 16, 16)] = lax.iota(jnp.int32, 16) + (base + j * 16)


def _tile_chunks(s, body):
    rem = NROWB - (NROWB // NS) * NS
    nch = jnp.where(s < rem, NROWB // NS + 1, NROWB // NS).astype(jnp.int32)
    def it(i, carry):
        body(s + i * NS)
        return carry
    lax.fori_loop(0, nch, it, 0, unroll=False)


NPAD = 10240


EPT1 = E // NC // NS
EPT2 = E // NS


NBUF = 3


def _edge_pipeline(tbl_hbm, src_hbm, dst_hbm, w_hbm, ebase,
                   srcb, dstb, wb, rows, semi, semg, sems,
                   acc, nblk, shift, cnt_v):
    one0 = jnp.where(lax.iota(jnp.int32, 16) == 0, 1.0, 0.0)

    def fire_idx(q, b):
        off = ebase + b * K
        pltpu.make_async_copy(src_hbm.at[pl.ds(off, K)], srcb[q],
                              semi[q]).start()
        pltpu.make_async_copy(dst_hbm.at[pl.ds(off, K)], dstb[q],
                              semi[q]).start()
        pltpu.make_async_copy(w_hbm.at[pl.ds(off, K)], wb[q], semi[q]).start()

    def wait_idx(q):
        off = ebase
        pltpu.make_async_copy(src_hbm.at[pl.ds(off, K)], srcb[q],
                              semi[q]).wait()
        pltpu.make_async_copy(dst_hbm.at[pl.ds(off, K)], dstb[q],
                              semi[q]).wait()
        pltpu.make_async_copy(w_hbm.at[pl.ds(off, K)], wb[q], semi[q]).wait()

    def prep_gather(q):
        if shift is not None:
            for j in range(K // 16):
                sl = pl.ds(j * 16, 16)
                srcb[q][sl] = srcb[q][sl] + shift
        pltpu.make_async_copy(tbl_hbm.at[srcb[q]], rows[q], semg[q]).start()

    def wait_gather(q):
        pltpu.make_async_copy(tbl_hbm.at[srcb[q]], rows[q], semg[q]).wait()

    def fire_scatter(q):
        pltpu.make_async_copy(rows[q], acc.at[dstb[q]], sems[q]).start(add=True)

    def wait_scatter(q):
        pltpu.make_async_copy(rows[q], acc.at[dstb[q]], sems[q]).wait()

    def step(q, b):
        f = b + NBUF - 1
        fq = (NBUF - 1 + q) % NBUF

        @pl.when(b >= 1)
        def _():
            wait_scatter(fq)

        @pl.when(f < nblk)
        def _():
            fire_idx(fq, f)
        if cnt_v is not None:
            def cnt_group(j, carry2):
                d16 = dstb[q][pl.ds(j * 16, 16)]
                for l in range(16):
                    d = d16[l]
                    cnt_v[pl.ds(d, 16)] = cnt_v[pl.ds(d, 16)] + one0
                return carry2
            lax.fori_loop(0, K // 16, cnt_group, 0, unroll=False)
        wait_gather(q)
        _scale_rows(rows[q], wb[q], D_IN // 16)

        @pl.when(f < nblk)
        def _():
            wait_idx(fq)
            prep_gather(fq)
        fire_scatter(q)

    for i in range(NBUF - 1):
        fire_idx(i, i)
        wait_idx(i)
        prep_gather(i)

    def group(p, carry):
        for b2 in range(NBUF):
            step(b2, NBUF * p + b2)
        return carry
    lax.fori_loop(0, nblk // NBUF, group, 0, unroll=False)
    for b in range(NBUF * (nblk // NBUF), nblk):
        step(b % NBUF, b)
    wait_scatter((nblk - 1) % NBUF)


@functools.partial(
    pl.kernel,
    out_type=[
        jax.ShapeDtypeStruct((NC * N, D_IN), jnp.float32),
        jax.ShapeDtypeStruct((NC * NS * NPAD,), jnp.float32),
    ],
    mesh=_MESH,
    scratch_types=[
        [pltpu.VMEM((K,), jnp.int32)] * 3,
        [pltpu.VMEM((K,), jnp.int32)] * 3,
        [pltpu.VMEM((K,), jnp.float32)] * 3,
        pltpu.VMEM((K,), jnp.int32),
        [pltpu.VMEM((K, D_IN), jnp.float32)] * 3,
        pltpu.VMEM((N + 16,), jnp.float32),
        pltpu.VMEM_SHARED((N, D_IN), jnp.float32),
        [pltpu.SemaphoreType.DMA] * 3,
        [pltpu.SemaphoreType.DMA] * 3,
        [pltpu.SemaphoreType.DMA] * 3,
        pltpu.SemaphoreType.DMA,
    ],
)
def _sc_agg1(x_hbm, src_hbm, dst_hbm, w_hbm,
             agg_out, cnt_out,
             srcb, dstb, wb, iota_v, rows, cnt_v, acc, semi, semg, sems, sem):
    c = lax.axis_index("c")
    s = lax.axis_index("s")
    _zero_fill(rows[0], K, D_IN // 16)

    def zero_cnt(i, carry):
        cnt_v[pl.ds(i * 16, 16)] = jnp.zeros((16,), jnp.float32)
        return carry
    lax.fori_loop(0, (N + 16) // 16, zero_cnt, 0, unroll=False)

    def init_chunk(ch):
        _fill_iota(iota_v, ch * K)
        pltpu.sync_copy(rows[0], acc.at[iota_v])
    _tile_chunks(s, init_chunk)
    plsc.subcore_barrier()

    _edge_pipeline(x_hbm, src_hbm, dst_hbm, w_hbm, (c * NS + s) * EPT1,
                   srcb, dstb, wb, rows, semi, semg, sems,
                   acc, EPT1 // K, None, cnt_v)
    plsc.subcore_barrier()

    def drain_chunk(ch):
        r0 = ch * K
        _fill_iota(iota_v, r0)
        pltpu.async_copy(acc.at[iota_v], rows[0], sem).wait()
        pltpu.sync_copy(rows[0], agg_out.at[pl.ds(c * N + r0, K)])
    _tile_chunks(s, drain_chunk)
    t = c * NS + s
    pltpu.sync_copy(cnt_v.at[pl.ds(0, N)], cnt_out.at[pl.ds(t * NPAD, N)])


@functools.partial(
    pl.kernel,
    out_type=jax.ShapeDtypeStruct((NC * N, D_IN), jnp.float32),
    mesh=_MESH,
    scratch_types=[
        [pltpu.VMEM((K,), jnp.int32)] * 3,
        [pltpu.VMEM((K,), jnp.int32)] * 3,
        [pltpu.VMEM((K,), jnp.float32)] * 3,
        pltpu.VMEM((K,), jnp.int32),
        [pltpu.VMEM((K, D_IN), jnp.float32)] * 3,
        pltpu.VMEM_SHARED((N, D_IN), jnp.float32),
        [pltpu.SemaphoreType.DMA] * 3,
        [pltpu.SemaphoreType.DMA] * 3,
        [pltpu.SemaphoreType.DMA] * 3,
        pltpu.SemaphoreType.DMA,
    ],
)
def _sc_agg2(zf_hbm, src_hbm, dst_hbm, w_hbm,
             agg_out,
             srcb, dstb, wb, iota_v, rows, acc, semi, semg, sems, sem):
    c = lax.axis_index("c")
    s = lax.axis_index("s")
    _zero_fill(rows[0], K, D_IN // 16)

    def init_chunk(ch):
        _fill_iota(iota_v, ch * K)
        pltpu.sync_copy(rows[0], acc.at[iota_v])
    _tile_chunks(s, init_chunk)
    plsc.subcore_barrier()

    _edge_pipeline(zf_hbm, src_hbm, dst_hbm, w_hbm, s * EPT2,
                   srcb, dstb, wb, rows, semi, semg, sems,
                   acc, EPT2 // K, c * N, None)
    plsc.subcore_barrier()

    def drain_chunk(ch):
        r0 = ch * K
        _fill_iota(iota_v, r0)
        pltpu.async_copy(acc.at[iota_v], rows[0], sem).wait()
        pltpu.sync_copy(rows[0], agg_out.at[pl.ds(c * N + r0, K)])
    _tile_chunks(s, drain_chunk)


def _tc_layer1_body(agg_ref, cnt_ref, x_ref, wl_ref, wr_ref, b_ref, h_ref):
    agg = agg_ref[0] + agg_ref[1]
    cnt = jnp.sum(cnt_ref[...], axis=1, keepdims=True)
    inv = 1.0 / jnp.maximum(cnt, 1.0)
    t = (jnp.dot(agg * inv, wl_ref[...], preferred_element_type=jnp.float32,
                 precision=lax.Precision.HIGHEST)
         + jnp.dot(x_ref[...], wr_ref[...], preferred_element_type=jnp.float32,
                 precision=lax.Precision.HIGHEST)
         + b_ref[...])
    h_ref[...] = jnp.maximum(t, 0.0)


def _tc_layer2_mm_body(h_ref, wl_ref, wr_ref, b_ref, z_ref, r_ref):
    h = h_ref[...]
    z_ref[0] = jnp.dot(h, wl_ref[...], preferred_element_type=jnp.float32,
                 precision=lax.Precision.HIGHEST)
    r_ref[...] = (jnp.dot(h, wr_ref[...], preferred_element_type=jnp.float32,
                 precision=lax.Precision.HIGHEST)
                  + b_ref[...])


def _tc_final_body(agg_ref, cnt_ref, r_ref, out_ref):
    cnt = jnp.sum(cnt_ref[...], axis=1, keepdims=True)
    inv = 1.0 / jnp.maximum(cnt, 1.0)
    out_ref[...] = agg_ref[0] * inv + r_ref[...]


def kernel(x, edge_index, edge_weight, W1l, W1r, b1, W2l, W2r, b2):
    src = edge_index[0].astype(jnp.int32)
    dst = edge_index[1].astype(jnp.int32)
    w = edge_weight.astype(jnp.float32)

    agg1_f, cnt_f = _sc_agg1(x, src, dst, w)
    agg1 = agg1_f.reshape(NC, N, D_IN)
    cnt = cnt_f.reshape(NC * NS, NPAD)[:, :N].T

    R = 1000
    grid1 = (N // R,)
    h = pl.pallas_call(
        _tc_layer1_body,
        grid=grid1,
        in_specs=[
            pl.BlockSpec((NC, R, D_IN), lambda i: (0, i, 0)),
            pl.BlockSpec((R, NC * NS), lambda i: (i, 0)),
            pl.BlockSpec((R, D_IN), lambda i: (i, 0)),
            pl.BlockSpec((D_IN, D_H1), lambda i: (0, 0)),
            pl.BlockSpec((D_IN, D_H1), lambda i: (0, 0)),
            pl.BlockSpec((1, D_H1), lambda i: (0, 0)),
        ],
        out_specs=pl.BlockSpec((R, D_H1), lambda i: (i, 0)),
        out_shape=jax.ShapeDtypeStruct((N, D_H1), jnp.float32),
    )(agg1, cnt, x, W1l, W1r, b1.reshape(1, D_H1))

    grid2 = (NC, N // R)
    z_split, r = pl.pallas_call(
        _tc_layer2_mm_body,
        grid=grid2,
        in_specs=[
            pl.BlockSpec((R, D_H1), lambda c, i: (i, 0)),
            pl.BlockSpec((D_H1, D_IN), lambda c, i: (0, c)),
            pl.BlockSpec((D_H1, D_IN), lambda c, i: (0, c)),
            pl.BlockSpec((1, D_IN), lambda c, i: (0, c)),
        ],
        out_specs=[
            pl.BlockSpec((1, R, D_IN), lambda c, i: (c, i, 0)),
            pl.BlockSpec((R, D_IN), lambda c, i: (i, c)),
        ],
        out_shape=[
            jax.ShapeDtypeStruct((NC, N, D_IN), jnp.float32),
            jax.ShapeDtypeStruct((N, D_H2), jnp.float32),
        ],
    )(h, W2l, W2r, b2.reshape(1, D_H2))

    agg2_f = _sc_agg2(z_split.reshape(NC * N, D_IN), src, dst, w)
    agg2 = agg2_f.reshape(NC, N, D_IN)

    out = pl.pallas_call(
        _tc_final_body,
        grid=grid2,
        in_specs=[
            pl.BlockSpec((1, R, D_IN), lambda c, i: (c, i, 0)),
            pl.BlockSpec((R, NC * NS), lambda c, i: (i, 0)),
            pl.BlockSpec((R, D_IN), lambda c, i: (i, c)),
        ],
        out_specs=pl.BlockSpec((R, D_IN), lambda c, i: (i, c)),
        out_shape=jax.ShapeDtypeStruct((N, D_H2), jnp.float32),
    )(agg2, cnt, r)
    return out


# speedup vs baseline: 1.2057x; 1.2057x over previous
import functools

import jax
import jax.numpy as jnp
from jax import lax
from jax.experimental import pallas as pl
from jax.experimental.pallas import tpu as pltpu
from jax.experimental.pallas import tpu_sc as plsc

N = 10000
E = 320000
D_IN = 128
D_H1 = 512
D_H2 = 256

NC = 2
NS = 16
K = 80
NROWB = N // K

_MESH = plsc.VectorSubcoreMesh(core_axis_name="c", subcore_axis_name="s")


def _scale_rows(rows_v, w_v, n_feat16):
    dnums = lax.GatherDimensionNumbers(
        offset_dims=(), collapsed_slice_dims=(0,), start_index_map=(0,))
    def group(j, carry):
        w16 = w_v[pl.ds(j * 16, 16)]
        for l in range(16):
            wk = lax.gather(w16, jnp.full((16, 1), l, jnp.int32), dnums, (1,),
                            mode=lax.GatherScatterMode.PROMISE_IN_BOUNDS)
            k = j * 16 + l
            for d in range(n_feat16):
                sl = pl.ds(d * 16, 16)
                rows_v[k, sl] = rows_v[k, sl] * wk
        return carry
    lax.fori_loop(0, K // 16, group, 0, unroll=False)


def _zero_fill(buf, nrows, width16):
    def row(i, carry):
        for d in range(width16):
            buf[i, pl.ds(d * 16, 16)] = jnp.zeros((16,), jnp.float32)
        return carry
    lax.fori_loop(0, nrows, row, 0, unroll=False)


def _fill_iota(idx_v, base):
    for j in range(K // 16):
        idx_v[pl.ds(j * 16, 16)] = lax.iota(jnp.int32, 16) + (base + j * 16)


def _tile_chunks(s, body):
    rem = NROWB - (NROWB // NS) * NS
    nch = jnp.where(s < rem, NROWB // NS + 1, NROWB // NS).astype(jnp.int32)
    def it(i, carry):
        body(s + i * NS)
        return carry
    lax.fori_loop(0, nch, it, 0, unroll=False)


NPAD = 10240


EPT1 = E // NC // NS
EPT2 = E // NS


def _edge_pipeline(tbl_hbm, src_hbm, dst_hbm, w_hbm, ebase,
                   srcb, dstb, wb, rows, semi, semg, sems,
                   acc, nblk, shift, cnt_v):
    one0 = jnp.where(lax.iota(jnp.int32, 16) == 0, 1.0, 0.0)
    nbuf = len(rows)

    def fire_idx(q, b):
        off = ebase + b * K
        pltpu.make_async_copy(src_hbm.at[pl.ds(off, K)], srcb[q],
                              semi[q]).start()
        pltpu.make_async_copy(dst_hbm.at[pl.ds(off, K)], dstb[q],
                              semi[q]).start()
        pltpu.make_async_copy(w_hbm.at[pl.ds(off, K)], wb[q], semi[q]).start()

    def wait_idx(q):
        off = ebase
        pltpu.make_async_copy(src_hbm.at[pl.ds(off, K)], srcb[q],
                              semi[q]).wait()
        pltpu.make_async_copy(dst_hbm.at[pl.ds(off, K)], dstb[q],
                              semi[q]).wait()
        pltpu.make_async_copy(w_hbm.at[pl.ds(off, K)], wb[q], semi[q]).wait()

    def prep_gather(q):
        if shift is not None:
            for j in range(K // 16):
                sl = pl.ds(j * 16, 16)
                srcb[q][sl] = srcb[q][sl] + shift
        pltpu.make_async_copy(tbl_hbm.at[srcb[q]], rows[q], semg[q]).start()

    def wait_gather(q):
        pltpu.make_async_copy(tbl_hbm.at[srcb[q]], rows[q], semg[q]).wait()

    def fire_scatter(q):
        pltpu.make_async_copy(rows[q], acc.at[dstb[q]], sems[q]).start(add=True)

    def wait_scatter(q):
        pltpu.make_async_copy(rows[q], acc.at[dstb[q]], sems[q]).wait()

    def step(q, b):
        f = b + nbuf - 1
        fq = (nbuf - 1 + q) % nbuf

        @pl.when(b >= 1)
        def _():
            wait_scatter(fq)

        @pl.when(f < nblk)
        def _():
            fire_idx(fq, f)
        if cnt_v is not None:
            def cnt_group(j, carry2):
                d16 = dstb[q][pl.ds(j * 16, 16)]
                for l in range(16):
                    d = d16[l]
                    cnt_v[pl.ds(d, 16)] = cnt_v[pl.ds(d, 16)] + one0
                return carry2
            lax.fori_loop(0, K // 16, cnt_group, 0, unroll=False)
        wait_gather(q)
        _scale_rows(rows[q], wb[q], D_IN // 16)

        @pl.when(f < nblk)
        def _():
            wait_idx(fq)
            prep_gather(fq)
        fire_scatter(q)

    for i in range(nbuf - 1):
        fire_idx(i, i)
        wait_idx(i)
        prep_gather(i)

    def group(p, carry):
        for b2 in range(nbuf):
            step(b2, nbuf * p + b2)
        return carry
    lax.fori_loop(0, nblk // nbuf, group, 0, unroll=False)
    for b in range(nbuf * (nblk // nbuf), nblk):
        step(b % nbuf, b)
    wait_scatter((nblk - 1) % nbuf)


@functools.partial(
    pl.kernel,
    out_type=[
        jax.ShapeDtypeStruct((NC * N, D_IN), jnp.float32),
        jax.ShapeDtypeStruct((NC * NS * NPAD,), jnp.float32),
    ],
    mesh=_MESH,
    scratch_types=[
        [pltpu.VMEM((K,), jnp.int32)] * 3,
        [pltpu.VMEM((K,), jnp.int32)] * 3,
        [pltpu.VMEM((K,), jnp.float32)] * 3,
        pltpu.VMEM((K,), jnp.int32),
        [pltpu.VMEM((K, D_IN), jnp.float32)] * 3,
        pltpu.VMEM((N + 16,), jnp.float32),
        pltpu.VMEM_SHARED((N, D_IN), jnp.float32),
        [pltpu.SemaphoreType.DMA] * 3,
        [pltpu.SemaphoreType.DMA] * 3,
        [pltpu.SemaphoreType.DMA] * 3,
        pltpu.SemaphoreType.DMA,
    ],
)
def _sc_agg1(x_hbm, src_hbm, dst_hbm, w_hbm,
             agg_out, cnt_out,
             srcb, dstb, wb, iota_v, rows, cnt_v, acc, semi, semg, sems, sem):
    c = lax.axis_index("c")
    s = lax.axis_index("s")
    _zero_fill(rows[0], K, D_IN // 16)

    def zero_cnt(i, carry):
        cnt_v[pl.ds(i * 16, 16)] = jnp.zeros((16,), jnp.float32)
        return carry
    lax.fori_loop(0, (N + 16) // 16, zero_cnt, 0, unroll=False)

    def init_chunk(ch):
        _fill_iota(iota_v, ch * K)
        pltpu.sync_copy(rows[0], acc.at[iota_v])
    _tile_chunks(s, init_chunk)
    plsc.subcore_barrier()

    _edge_pipeline(x_hbm, src_hbm, dst_hbm, w_hbm, (c * NS + s) * EPT1,
                   srcb, dstb, wb, rows, semi, semg, sems,
                   acc, EPT1 // K, None, cnt_v)
    plsc.subcore_barrier()

    def drain_chunk(ch):
        r0 = ch * K
        _fill_iota(iota_v, r0)
        pltpu.async_copy(acc.at[iota_v], rows[0], sem).wait()
        pltpu.sync_copy(rows[0], agg_out.at[pl.ds(c * N + r0, K)])
    _tile_chunks(s, drain_chunk)
    t = c * NS + s
    pltpu.sync_copy(cnt_v.at[pl.ds(0, N)], cnt_out.at[pl.ds(t * NPAD, N)])


@functools.partial(
    pl.kernel,
    out_type=jax.ShapeDtypeStruct((NC * N, D_IN), jnp.float32),
    mesh=_MESH,
    scratch_types=[
        [pltpu.VMEM((K,), jnp.int32)] * 4,
        [pltpu.VMEM((K,), jnp.int32)] * 4,
        [pltpu.VMEM((K,), jnp.float32)] * 4,
        pltpu.VMEM((K,), jnp.int32),
        [pltpu.VMEM((K, D_IN), jnp.float32)] * 4,
        pltpu.VMEM_SHARED((N, D_IN), jnp.float32),
        [pltpu.SemaphoreType.DMA] * 4,
        [pltpu.SemaphoreType.DMA] * 4,
        [pltpu.SemaphoreType.DMA] * 4,
        pltpu.SemaphoreType.DMA,
    ],
)
def _sc_agg2(zf_hbm, src_hbm, dst_hbm, w_hbm,
             agg_out,
             srcb, dstb, wb, iota_v, rows, acc, semi, semg, sems, sem):
    c = lax.axis_index("c")
    s = lax.axis_index("s")
    _zero_fill(rows[0], K, D_IN // 16)

    def init_chunk(ch):
        _fill_iota(iota_v, ch * K)
        pltpu.sync_copy(rows[0], acc.at[iota_v])
    _tile_chunks(s, init_chunk)
    plsc.subcore_barrier()

    _edge_pipeline(zf_hbm, src_hbm, dst_hbm, w_hbm, s * EPT2,
                   srcb, dstb, wb, rows, semi, semg, sems,
                   acc, EPT2 // K, c * N, None)
    plsc.subcore_barrier()

    def drain_chunk(ch):
        r0 = ch * K
        _fill_iota(iota_v, r0)
        pltpu.async_copy(acc.at[iota_v], rows[0], sem).wait()
        pltpu.sync_copy(rows[0], agg_out.at[pl.ds(c * N + r0, K)])
    _tile_chunks(s, drain_chunk)


def _tc_fused_body(agg_ref, cnt_ref, x_ref, w1l_ref, w1r_ref, b1_ref,
                   w2l_ref, w2r_ref, b2_ref, z_ref, r_ref):
    agg = agg_ref[0] + agg_ref[1]
    cnt = jnp.sum(cnt_ref[...], axis=1, keepdims=True)
    inv = 1.0 / jnp.maximum(cnt, 1.0)
    t = (jnp.dot(agg * inv, w1l_ref[...], preferred_element_type=jnp.float32,
                 precision=lax.Precision.HIGHEST)
         + jnp.dot(x_ref[...], w1r_ref[...], preferred_element_type=jnp.float32,
                   precision=lax.Precision.HIGHEST)
         + b1_ref[...])
    h = jnp.maximum(t, 0.0)
    z = jnp.dot(h, w2l_ref[...], preferred_element_type=jnp.float32,
                precision=lax.Precision.HIGHEST)
    z_ref[0] = z[:, :D_IN]
    z_ref[1] = z[:, D_IN:]
    r_ref[...] = (jnp.dot(h, w2r_ref[...], preferred_element_type=jnp.float32,
                          precision=lax.Precision.HIGHEST)
                  + b2_ref[...])


def _tc_final_body(agg_ref, cnt_ref, r_ref, out_ref):
    cnt = jnp.sum(cnt_ref[...], axis=1, keepdims=True)
    inv = 1.0 / jnp.maximum(cnt, 1.0)
    out_ref[...] = agg_ref[0] * inv + r_ref[...]


def kernel(x, edge_index, edge_weight, W1l, W1r, b1, W2l, W2r, b2):
    src = edge_index[0].astype(jnp.int32)
    dst = edge_index[1].astype(jnp.int32)
    w = edge_weight.astype(jnp.float32)

    agg1_f, cnt_f = _sc_agg1(x, src, dst, w)
    agg1 = agg1_f.reshape(NC, N, D_IN)
    cnt = cnt_f.reshape(NC * NS, NPAD)[:, :N].T

    R = 1000
    grid1 = (N // R,)
    z_split, r = pl.pallas_call(
        _tc_fused_body,
        grid=grid1,
        in_specs=[
            pl.BlockSpec((NC, R, D_IN), lambda i: (0, i, 0)),
            pl.BlockSpec((R, NC * NS), lambda i: (i, 0)),
            pl.BlockSpec((R, D_IN), lambda i: (i, 0)),
            pl.BlockSpec((D_IN, D_H1), lambda i: (0, 0)),
            pl.BlockSpec((D_IN, D_H1), lambda i: (0, 0)),
            pl.BlockSpec((1, D_H1), lambda i: (0, 0)),
            pl.BlockSpec((D_H1, D_H2), lambda i: (0, 0)),
            pl.BlockSpec((D_H1, D_H2), lambda i: (0, 0)),
            pl.BlockSpec((1, D_H2), lambda i: (0, 0)),
        ],
        out_specs=[
            pl.BlockSpec((NC, R, D_IN), lambda i: (0, i, 0)),
            pl.BlockSpec((R, D_H2), lambda i: (i, 0)),
        ],
        out_shape=[
            jax.ShapeDtypeStruct((NC, N, D_IN), jnp.float32),
            jax.ShapeDtypeStruct((N, D_H2), jnp.float32),
        ],
    )(agg1, cnt, x, W1l, W1r, b1.reshape(1, D_H1), W2l, W2r,
      b2.reshape(1, D_H2))

    agg2_f = _sc_agg2(z_split.reshape(NC * N, D_IN), src, dst, w)
    agg2 = agg2_f.reshape(NC, N, D_IN)

    grid2 = (NC, N // R)
    out = pl.pallas_call(
        _tc_final_body,
        grid=grid2,
        in_specs=[
            pl.BlockSpec((1, R, D_IN), lambda c, i: (c, i, 0)),
            pl.BlockSpec((R, NC * NS), lambda c, i: (i, 0)),
            pl.BlockSpec((R, D_IN), lambda c, i: (i, c)),
        ],
        out_specs=pl.BlockSpec((R, D_IN), lambda c, i: (i, c)),
        out_shape=jax.ShapeDtypeStruct((N, D_H2), jnp.float32),
    )(agg2, cnt, r)
    return out
